# Initial kernel scaffold; baseline (speedup 1.0000x reference)
import jax
import jax.numpy as jnp
from jax.experimental import pallas as pl


def kernel(moe_inp, Wg, bg, w1, b1, w2, b2):
    raise NotImplementedError("write your pallas kernel here")


# speedup vs baseline: 2.4899x; 2.4899x over previous
import functools

import jax
import jax.numpy as jnp
from jax.experimental import pallas as pl
from jax.experimental.pallas import tpu as pltpu


def _moe_body(blk_e_ref, x_ref, w1_ref, b1_ref, w2_ref, b2_ref,
              rows_ref, gate_ref, out_ref, xb_ref, acc_ref, *, n_j, blk_b):
    g = pl.program_id(0)
    j = pl.program_id(1)
    t_tot = x_ref.shape[0]

    @pl.when(j == 0)
    def _gather():
        rows = rows_ref[0, 0, :]
        eq = jax.lax.broadcasted_iota(jnp.int32, (blk_b, t_tot), 1) == rows[:, None]
        a = eq.astype(jnp.float32)
        xb_ref[...] = jnp.dot(a, x_ref[...], preferred_element_type=jnp.float32)

    h = jnp.dot(xb_ref[...], w1_ref[0], preferred_element_type=jnp.float32)
    h = jnp.maximum(h + b1_ref[0], 0.0)
    contrib = jnp.dot(h, w2_ref[0], preferred_element_type=jnp.float32)

    @pl.when(j == 0)
    def _init_acc():
        acc_ref[...] = contrib + b2_ref[0]

    @pl.when(j > 0)
    def _add_acc():
        acc_ref[...] += contrib

    @pl.when(j == n_j - 1)
    def _scatter():
        rows = rows_ref[0, 0, :]
        gw = gate_ref[0, 0, :]
        eq_t = jax.lax.broadcasted_iota(jnp.int32, (t_tot, blk_b), 0) == rows[None, :]
        aw_t = jnp.where(eq_t, gw[None, :], 0.0)
        contrib_out = jnp.dot(aw_t, acc_ref[...], preferred_element_type=jnp.float32)

        @pl.when(g == 0)
        def _():
            out_ref[...] = contrib_out

        @pl.when(g > 0)
        def _():
            out_ref[...] += contrib_out


def kernel(moe_inp, Wg, bg, w1, b1, w2, b2):
    x = moe_inp
    t, d = x.shape
    e, _, dff = w1.shape
    k = 2
    tk = t * k
    blk_b = min(256, tk)
    n_g = tk // blk_b + e
    f = min(1024, dff)
    n_j = dff // f

    logits = x @ Wg + bg
    topv, topi = jax.lax.top_k(logits, k)
    gate = jax.nn.softmax(topv, axis=-1)
    flat_idx = topi.reshape(-1)
    order = jnp.argsort(flat_idx)
    counts = jnp.bincount(flat_idx, length=e)
    starts = jnp.concatenate([jnp.zeros((1,), jnp.int32),
                              jnp.cumsum(counts).astype(jnp.int32)])[:e]
    nb = (counts + blk_b - 1) // blk_b
    nb_csum = jnp.cumsum(nb)
    first_blk = jnp.concatenate([jnp.zeros((1,), jnp.int32),
                                 nb_csum.astype(jnp.int32)])[:e]
    gidx = jnp.arange(n_g)
    blk_e = jnp.clip(jnp.searchsorted(nb_csum, gidx, side="right"), 0, e - 1)
    offs = gidx - first_blk[blk_e]
    pos = starts[blk_e][:, None] + offs[:, None] * blk_b + jnp.arange(blk_b)[None, :]
    valid = (pos < (starts + counts.astype(jnp.int32))[blk_e][:, None]) & (offs[:, None] >= 0)
    posc = jnp.clip(pos, 0, tk - 1)
    slot = order[posc]
    rows = jnp.where(valid, slot // k, 0).astype(jnp.int32)
    gatew = jnp.where(valid, gate.reshape(-1)[slot], 0.0).astype(jnp.float32)

    rows3 = rows.reshape(n_g, 1, blk_b)
    gate3 = gatew.reshape(n_g, 1, blk_b)
    b1r = b1.reshape(e, 1, dff)
    b2r = b2.reshape(e, 1, d)

    grid_spec = pltpu.PrefetchScalarGridSpec(
        num_scalar_prefetch=1,
        grid=(n_g, n_j),
        in_specs=[
            pl.BlockSpec((t, d), lambda g, j, be: (0, 0)),
            pl.BlockSpec((1, d, f), lambda g, j, be: (be[g], 0, j)),
            pl.BlockSpec((1, 1, f), lambda g, j, be: (be[g], 0, j)),
            pl.BlockSpec((1, f, d), lambda g, j, be: (be[g], j, 0)),
            pl.BlockSpec((1, 1, d), lambda g, j, be: (be[g], 0, 0)),
            pl.BlockSpec((1, 1, blk_b), lambda g, j, be: (g, 0, 0)),
            pl.BlockSpec((1, 1, blk_b), lambda g, j, be: (g, 0, 0)),
        ],
        out_specs=pl.BlockSpec((t, d), lambda g, j, be: (0, 0)),
        scratch_shapes=[
            pltpu.VMEM((blk_b, d), jnp.float32),
            pltpu.VMEM((blk_b, d), jnp.float32),
        ],
    )

    out = pl.pallas_call(
        functools.partial(_moe_body, n_j=n_j, blk_b=blk_b),
        grid_spec=grid_spec,
        out_shape=jax.ShapeDtypeStruct((t, d), jnp.float32),
        compiler_params=pltpu.CompilerParams(
            dimension_semantics=("arbitrary", "arbitrary"),
        ),
    )(blk_e.astype(jnp.int32), x, w1, b1r, w2, b2r, rows3, gate3)
    return out
